# baseline (speedup 1.0000x reference)
import functools

import jax
import jax.numpy as jnp
from jax.experimental import pallas as pl
from jax.experimental.pallas import tpu as pltpu


def _gcn_block_kernel(edges_ref, ev_ref, hp_ref, nodes_ref, w_ref, out_ref,
                      g_ref, *, n, e, out_f):
    @pl.when(pl.program_id(0) == 0)
    def _init():
        nt = jnp.dot(nodes_ref[:], w_ref[:], preferred_element_type=jnp.float32)
        for i in range(n):
            g_ref[:, i * out_f:(i + 1) * out_f] = jnp.dot(
                hp_ref[i * e:(i + 1) * e, :], nt,
                preferred_element_type=jnp.float32)

    ew = jnp.sum(edges_ref[:] * ev_ref[0][None, None, :], axis=-1)
    out_ref[:] = jnp.dot(ew, g_ref[:], preferred_element_type=jnp.float32)


def kernel(nodes, edges, weight_matrix, edge_weight_vec, adj_matrix, inc_matrix):
    b, e, k = edges.shape
    n, in_f = nodes.shape
    out_f = weight_matrix.shape[1]
    f32 = jnp.float32

    adj_sl = adj_matrix + jnp.eye(n, dtype=adj_matrix.dtype)
    deg = jnp.sum(adj_sl, axis=1)
    d_inv = 1.0 / jnp.sqrt(deg)
    lap = d_inv[:, None] * adj_sl * d_inv[None, :]
    mask = ((inc_matrix[:, None, :] * inc_matrix[None, :, :]) != 0).astype(f32)
    hp = (mask * lap[:, :, None]).transpose(0, 2, 1).reshape(n * e, n)
    ev2 = edge_weight_vec.astype(f32).reshape(1, k)

    bb = 256
    out = pl.pallas_call(
        functools.partial(_gcn_block_kernel, n=n, e=e, out_f=out_f),
        grid=(b // bb,),
        in_specs=[
            pl.BlockSpec((bb, e, k), lambda i: (i, 0, 0)),
            pl.BlockSpec((1, k), lambda i: (0, 0)),
            pl.BlockSpec((n * e, n), lambda i: (0, 0)),
            pl.BlockSpec((n, in_f), lambda i: (0, 0)),
            pl.BlockSpec((in_f, out_f), lambda i: (0, 0)),
        ],
        out_specs=pl.BlockSpec((bb, n * out_f), lambda i: (i, 0)),
        out_shape=jax.ShapeDtypeStruct((b, n * out_f), f32),
        scratch_shapes=[pltpu.VMEM((e, n * out_f), f32)],
    )(edges, ev2, hp, nodes, weight_matrix)
    return out.reshape(b, n, out_f)


# speedup vs baseline: 1.8160x; 1.8160x over previous
import functools

import jax
import jax.numpy as jnp
from jax.experimental import pallas as pl
from jax.experimental.pallas import tpu as pltpu


def _tc_kernel(edges_ref, p_ref, hp_ref, nodes_ref, w_ref, out_ref, g_ref,
               *, n, e, out_f):
    @pl.when(pl.program_id(0) == 0)
    def _init():
        nt = jnp.dot(nodes_ref[:], w_ref[:], preferred_element_type=jnp.float32)
        for i in range(n):
            g_ref[:, i * out_f:(i + 1) * out_f] = jnp.dot(
                hp_ref[i * e:(i + 1) * e, :], nt,
                preferred_element_type=jnp.float32)

    ew = jnp.dot(edges_ref[:], p_ref[:], preferred_element_type=jnp.float32)
    for i in range(n):
        out_ref[:, i, :] = jnp.dot(ew, g_ref[:, i * out_f:(i + 1) * out_f],
                                   preferred_element_type=jnp.float32)


def _tc_kernel_aliased(acc_ref, edges_ref, p_ref, hp_ref, nodes_ref, w_ref,
                       out_ref, g_ref, *, n, e, out_f):
    del acc_ref
    _tc_kernel(edges_ref, p_ref, hp_ref, nodes_ref, w_ref, out_ref, g_ref,
               n=n, e=e, out_f=out_f)


def kernel(nodes, edges, weight_matrix, edge_weight_vec, adj_matrix, inc_matrix):
    b, e, k = edges.shape
    n, in_f = nodes.shape
    out_f = weight_matrix.shape[1]
    f32 = jnp.float32

    adj_sl = adj_matrix + jnp.eye(n, dtype=adj_matrix.dtype)
    deg = jnp.sum(adj_sl, axis=1)
    d_inv = 1.0 / jnp.sqrt(deg)
    lap = d_inv[:, None] * adj_sl * d_inv[None, :]
    mask = ((inc_matrix[:, None, :] * inc_matrix[None, :, :]) != 0).astype(f32)
    hp = (mask * lap[:, :, None]).transpose(0, 2, 1).reshape(n * e, n)
    p = jnp.kron(jnp.eye(e, dtype=f32), edge_weight_vec.astype(f32)[:, None])

    n_chunks = 4
    bc = b // n_chunks
    bb = 256
    spc = bc // bb

    small_specs = [
        pl.BlockSpec((e * k, e), lambda i: (0, 0)),
        pl.BlockSpec((n * e, n), lambda i: (0, 0)),
        pl.BlockSpec((n, in_f), lambda i: (0, 0)),
        pl.BlockSpec((in_f, out_f), lambda i: (0, 0)),
    ]
    out_shape = jax.ShapeDtypeStruct((b, n, out_f), f32)
    scratch = [pltpu.VMEM((e, n * out_f), f32)]

    def chunk_flat(c):
        return edges[c * bc:(c + 1) * bc].reshape(bc, e * k)

    def out_spec(c):
        return pl.BlockSpec((bb, n, out_f), lambda i, c=c: (c * spc + i, 0, 0))

    out = pl.pallas_call(
        functools.partial(_tc_kernel, n=n, e=e, out_f=out_f),
        grid=(spc,),
        in_specs=[pl.BlockSpec((bb, e * k), lambda i: (i, 0))] + small_specs,
        out_specs=out_spec(0),
        out_shape=out_shape,
        scratch_shapes=scratch,
    )(chunk_flat(0), p, hp, nodes, weight_matrix)

    for c in range(1, n_chunks):
        out = pl.pallas_call(
            functools.partial(_tc_kernel_aliased, n=n, e=e, out_f=out_f),
            grid=(spc,),
            in_specs=[pl.BlockSpec((8, 8, out_f), lambda i: (0, 0, 0)),
                      pl.BlockSpec((bb, e * k), lambda i: (i, 0))] + small_specs,
            out_specs=out_spec(c),
            out_shape=out_shape,
            scratch_shapes=scratch,
            input_output_aliases={0: 0},
        )(out, chunk_flat(c), p, hp, nodes, weight_matrix)
    return out


# speedup vs baseline: 1.9839x; 1.0925x over previous
import functools

import jax
import jax.numpy as jnp
from jax.experimental import pallas as pl
from jax.experimental.pallas import tpu as pltpu


def _tc_kernel(edges_ref, p_ref, hp_ref, nodes_ref, w_ref, out_ref, g_ref,
               *, n, e, out_f):
    @pl.when(pl.program_id(0) == 0)
    def _init():
        nt = jnp.dot(nodes_ref[:], w_ref[:], preferred_element_type=jnp.float32)
        for i in range(n):
            g_ref[:, i * out_f:(i + 1) * out_f] = jnp.dot(
                hp_ref[i * e:(i + 1) * e, :], nt,
                preferred_element_type=jnp.float32)

    ew = jnp.dot(edges_ref[:], p_ref[:], preferred_element_type=jnp.float32)
    for i in range(n):
        out_ref[:, i, :] = jnp.dot(ew, g_ref[:, i * out_f:(i + 1) * out_f],
                                   preferred_element_type=jnp.float32)


def _tc_kernel_aliased(acc_ref, edges_ref, p_ref, hp_ref, nodes_ref, w_ref,
                       out_ref, g_ref, *, n, e, out_f):
    del acc_ref
    _tc_kernel(edges_ref, p_ref, hp_ref, nodes_ref, w_ref, out_ref, g_ref,
               n=n, e=e, out_f=out_f)


def kernel(nodes, edges, weight_matrix, edge_weight_vec, adj_matrix, inc_matrix):
    b, e, k = edges.shape
    n, in_f = nodes.shape
    out_f = weight_matrix.shape[1]
    f32 = jnp.float32

    adj_sl = adj_matrix + jnp.eye(n, dtype=adj_matrix.dtype)
    deg = jnp.sum(adj_sl, axis=1)
    d_inv = 1.0 / jnp.sqrt(deg)
    lap = d_inv[:, None] * adj_sl * d_inv[None, :]
    mask = ((inc_matrix[:, None, :] * inc_matrix[None, :, :]) != 0).astype(f32)
    hp = (mask * lap[:, :, None]).transpose(0, 2, 1).reshape(n * e, n)
    p = jnp.kron(jnp.eye(e, dtype=f32), edge_weight_vec.astype(f32)[:, None])

    n_chunks = 2
    bc = b // n_chunks
    bb = 256
    spc = bc // bb

    small_specs = [
        pl.BlockSpec((e * k, e), lambda i: (0, 0)),
        pl.BlockSpec((n * e, n), lambda i: (0, 0)),
        pl.BlockSpec((n, in_f), lambda i: (0, 0)),
        pl.BlockSpec((in_f, out_f), lambda i: (0, 0)),
    ]
    out_shape = jax.ShapeDtypeStruct((b, n, out_f), f32)
    scratch = [pltpu.VMEM((e, n * out_f), f32)]

    def chunk_flat(c):
        return edges[c * bc:(c + 1) * bc].reshape(bc, e * k)

    def out_spec(c):
        return pl.BlockSpec((bb, n, out_f), lambda i, c=c: (c * spc + i, 0, 0))

    out = pl.pallas_call(
        functools.partial(_tc_kernel, n=n, e=e, out_f=out_f),
        grid=(spc,),
        in_specs=[pl.BlockSpec((bb, e * k), lambda i: (i, 0))] + small_specs,
        out_specs=out_spec(0),
        out_shape=out_shape,
        scratch_shapes=scratch,
    )(chunk_flat(0), p, hp, nodes, weight_matrix)

    for c in range(1, n_chunks):
        out = pl.pallas_call(
            functools.partial(_tc_kernel_aliased, n=n, e=e, out_f=out_f),
            grid=(spc,),
            in_specs=[pl.BlockSpec((8, 8, out_f), lambda i: (0, 0, 0)),
                      pl.BlockSpec((bb, e * k), lambda i: (i, 0))] + small_specs,
            out_specs=out_spec(c),
            out_shape=out_shape,
            scratch_shapes=scratch,
            input_output_aliases={0: 0},
        )(out, chunk_flat(c), p, hp, nodes, weight_matrix)
    return out


# speedup vs baseline: 2.3597x; 1.1894x over previous
import functools

import jax
import jax.numpy as jnp
from jax.experimental import pallas as pl
from jax.experimental.pallas import tpu as pltpu


def _tc_kernel(edges_ref, p_ref, hp_ref, nodes_ref, w_ref, out_ref, g_ref,
               *, n, e, out_f):
    @pl.when(pl.program_id(0) == 0)
    def _init():
        nt = jnp.dot(nodes_ref[:], w_ref[:], preferred_element_type=jnp.float32)
        for i in range(n):
            g_ref[:, i * out_f:(i + 1) * out_f] = jnp.dot(
                hp_ref[i * e:(i + 1) * e, :], nt,
                preferred_element_type=jnp.float32)

    ew = jnp.dot(edges_ref[:], p_ref[:], preferred_element_type=jnp.float32)
    for i in range(n):
        out_ref[:, i, :] = jnp.dot(ew, g_ref[:, i * out_f:(i + 1) * out_f],
                                   preferred_element_type=jnp.float32)


def _tc_kernel_aliased(acc_ref, edges_ref, p_ref, hp_ref, nodes_ref, w_ref,
                       out_ref, g_ref, *, n, e, out_f):
    del acc_ref
    _tc_kernel(edges_ref, p_ref, hp_ref, nodes_ref, w_ref, out_ref, g_ref,
               n=n, e=e, out_f=out_f)


def kernel(nodes, edges, weight_matrix, edge_weight_vec, adj_matrix, inc_matrix):
    b, e, k = edges.shape
    n, in_f = nodes.shape
    out_f = weight_matrix.shape[1]
    f32 = jnp.float32

    adj_sl = adj_matrix + jnp.eye(n, dtype=adj_matrix.dtype)
    deg = jnp.sum(adj_sl, axis=1)
    d_inv = 1.0 / jnp.sqrt(deg)
    lap = d_inv[:, None] * adj_sl * d_inv[None, :]
    mask = ((inc_matrix[:, None, :] * inc_matrix[None, :, :]) != 0).astype(f32)
    hp = (mask * lap[:, :, None]).transpose(0, 2, 1).reshape(n * e, n)
    p = jnp.kron(jnp.eye(e, dtype=f32), edge_weight_vec.astype(f32)[:, None])

    n_chunks = 1
    bc = b // n_chunks
    bb = 256
    spc = bc // bb

    small_specs = [
        pl.BlockSpec((e * k, e), lambda i: (0, 0)),
        pl.BlockSpec((n * e, n), lambda i: (0, 0)),
        pl.BlockSpec((n, in_f), lambda i: (0, 0)),
        pl.BlockSpec((in_f, out_f), lambda i: (0, 0)),
    ]
    out_shape = jax.ShapeDtypeStruct((b, n, out_f), f32)
    scratch = [pltpu.VMEM((e, n * out_f), f32)]

    def chunk_flat(c):
        return edges[c * bc:(c + 1) * bc].reshape(bc, e * k)

    def out_spec(c):
        return pl.BlockSpec((bb, n, out_f), lambda i, c=c: (c * spc + i, 0, 0))

    out = pl.pallas_call(
        functools.partial(_tc_kernel, n=n, e=e, out_f=out_f),
        grid=(spc,),
        in_specs=[pl.BlockSpec((bb, e * k), lambda i: (i, 0))] + small_specs,
        out_specs=out_spec(0),
        out_shape=out_shape,
        scratch_shapes=scratch,
    )(chunk_flat(0), p, hp, nodes, weight_matrix)

    for c in range(1, n_chunks):
        out = pl.pallas_call(
            functools.partial(_tc_kernel_aliased, n=n, e=e, out_f=out_f),
            grid=(spc,),
            in_specs=[pl.BlockSpec((8, 8, out_f), lambda i: (0, 0, 0)),
                      pl.BlockSpec((bb, e * k), lambda i: (i, 0))] + small_specs,
            out_specs=out_spec(c),
            out_shape=out_shape,
            scratch_shapes=scratch,
            input_output_aliases={0: 0},
        )(out, chunk_flat(c), p, hp, nodes, weight_matrix)
    return out


# speedup vs baseline: 2.3909x; 1.0132x over previous
import functools

import jax
import jax.numpy as jnp
from jax.experimental import pallas as pl
from jax.experimental.pallas import tpu as pltpu


def _tc_kernel(edges_ref, p_ref, hp_ref, nodes_ref, w_ref, out_ref, g_ref,
               *, n, e, out_f):
    @pl.when(pl.program_id(0) == 0)
    def _init():
        nt = jnp.dot(nodes_ref[:], w_ref[:], preferred_element_type=jnp.float32)
        for i in range(n):
            g_ref[:, i * out_f:(i + 1) * out_f] = jnp.dot(
                hp_ref[i * e:(i + 1) * e, :], nt,
                preferred_element_type=jnp.float32)

    ew = jnp.dot(edges_ref[:], p_ref[:], preferred_element_type=jnp.float32)
    for i in range(n):
        out_ref[:, i, :] = jnp.dot(ew, g_ref[:, i * out_f:(i + 1) * out_f],
                                   preferred_element_type=jnp.float32)


def kernel(nodes, edges, weight_matrix, edge_weight_vec, adj_matrix, inc_matrix):
    b, e, k = edges.shape
    n, in_f = nodes.shape
    out_f = weight_matrix.shape[1]
    f32 = jnp.float32

    adj_sl = adj_matrix + jnp.eye(n, dtype=adj_matrix.dtype)
    deg = jnp.sum(adj_sl, axis=1)
    d_inv = 1.0 / jnp.sqrt(deg)
    lap = d_inv[:, None] * adj_sl * d_inv[None, :]
    mask = ((inc_matrix[:, None, :] * inc_matrix[None, :, :]) != 0).astype(f32)
    hp = (mask * lap[:, :, None]).transpose(0, 2, 1).reshape(n * e, n)
    p = jnp.kron(jnp.eye(e, dtype=f32), edge_weight_vec.astype(f32)[:, None])
    edges_flat = edges.reshape(b, e * k)

    bb = 512
    out = pl.pallas_call(
        functools.partial(_tc_kernel, n=n, e=e, out_f=out_f),
        grid=(b // bb,),
        in_specs=[
            pl.BlockSpec((bb, e * k), lambda i: (i, 0)),
            pl.BlockSpec((e * k, e), lambda i: (0, 0)),
            pl.BlockSpec((n * e, n), lambda i: (0, 0)),
            pl.BlockSpec((n, in_f), lambda i: (0, 0)),
            pl.BlockSpec((in_f, out_f), lambda i: (0, 0)),
        ],
        out_specs=pl.BlockSpec((bb, n, out_f), lambda i: (i, 0, 0)),
        out_shape=jax.ShapeDtypeStruct((b, n, out_f), f32),
        scratch_shapes=[pltpu.VMEM((e, n * out_f), f32)],
    )(edges_flat, p, hp, nodes, weight_matrix)
    return out


# speedup vs baseline: 3.7462x; 1.5669x over previous
import functools

import jax
import jax.numpy as jnp
from jax.experimental import pallas as pl
from jax.experimental.pallas import tpu as pltpu


def _tc_kernel(edges_ref, p_ref, hp_ref, nodes_ref, w_ref, out_ref, g_ref,
               *, n, e, out_f):
    @pl.when(pl.program_id(0) == 0)
    def _init():
        nt = jnp.dot(nodes_ref[:], w_ref[:], preferred_element_type=jnp.float32)
        for i in range(n):
            g_ref[:, i * out_f:(i + 1) * out_f] = jnp.dot(
                hp_ref[i * e:(i + 1) * e, :], nt,
                preferred_element_type=jnp.float32)

    ew = jnp.dot(edges_ref[:], p_ref[:], preferred_element_type=jnp.float32)
    res = jnp.dot(ew, g_ref[:], preferred_element_type=jnp.float32)
    out_ref[:] = res.reshape(res.shape[0], n, out_f)


def kernel(nodes, edges, weight_matrix, edge_weight_vec, adj_matrix, inc_matrix):
    b, e, k = edges.shape
    n, in_f = nodes.shape
    out_f = weight_matrix.shape[1]
    f32 = jnp.float32

    adj_sl = adj_matrix + jnp.eye(n, dtype=adj_matrix.dtype)
    deg = jnp.sum(adj_sl, axis=1)
    d_inv = 1.0 / jnp.sqrt(deg)
    lap = d_inv[:, None] * adj_sl * d_inv[None, :]
    mask = ((inc_matrix[:, None, :] * inc_matrix[None, :, :]) != 0).astype(f32)
    hp = (mask * lap[:, :, None]).transpose(0, 2, 1).reshape(n * e, n)
    p = jnp.kron(jnp.eye(e, dtype=f32), edge_weight_vec.astype(f32)[:, None])
    edges_flat = edges.reshape(b, e * k)

    bb = 512
    out = pl.pallas_call(
        functools.partial(_tc_kernel, n=n, e=e, out_f=out_f),
        grid=(b // bb,),
        in_specs=[
            pl.BlockSpec((bb, e * k), lambda i: (i, 0)),
            pl.BlockSpec((e * k, e), lambda i: (0, 0)),
            pl.BlockSpec((n * e, n), lambda i: (0, 0)),
            pl.BlockSpec((n, in_f), lambda i: (0, 0)),
            pl.BlockSpec((in_f, out_f), lambda i: (0, 0)),
        ],
        out_specs=pl.BlockSpec((bb, n, out_f), lambda i: (i, 0, 0)),
        out_shape=jax.ShapeDtypeStruct((b, n, out_f), f32),
        scratch_shapes=[pltpu.VMEM((e, n * out_f), f32)],
    )(edges_flat, p, hp, nodes, weight_matrix)
    return out
